# Initial kernel scaffold; baseline (speedup 1.0000x reference)
import jax
import jax.numpy as jnp
from jax.experimental import pallas as pl


def kernel(x, edge_index, W1, b1, W2, b2):
    raise NotImplementedError("write your pallas kernel here")


# speedup vs baseline: 68.1440x; 68.1440x over previous
import functools

import jax
import jax.numpy as jnp
from jax import lax
from jax.experimental import pallas as pl
from jax.experimental.pallas import tpu as pltpu
from jax.experimental.pallas import tpu_sc as plsc

N = 50000
E = 3200000
IN_F = 1433
HID = 16
CLS = 7

NC, NS = 2, 16
NW = NC * NS
RPT = 3200
NPAD = NS * RPT
STREAM = 128
SPC = 8
CHUNK = STREAM * SPC
EP = -(-E // (NW * CHUNK)) * (NW * CHUNK)
GROUPS = EP // STREAM
GPW = GROUPS // NW
CPW = EP // (NW * CHUNK)

_MESH = plsc.VectorSubcoreMesh(core_axis_name="c", subcore_axis_name="s")
_SC_PARAMS = pltpu.CompilerParams(use_tc_tiling_on_sc=False,
                                  internal_scratch_in_bytes=256 * 1024)


@functools.partial(
    pl.kernel,
    out_type=jax.ShapeDtypeStruct((NC * NPAD,), jnp.float32),
    mesh=_MESH,
    compiler_params=_SC_PARAMS,
    scratch_types=[
        pltpu.VMEM((SPC, STREAM), jnp.int32),
        pltpu.VMEM((STREAM,), jnp.float32),
        pltpu.VMEM((RPT,), jnp.float32),
        pltpu.VMEM_SHARED((NPAD,), jnp.float32),
    ],
)
def _deg_kernel(dst_hbm, zeros_hbm, out_hbm, didx, ones_v, tmp, acc):
    c = lax.axis_index("c")
    s = lax.axis_index("s")
    w = c * NS + s
    base = s * RPT
    for i in range(STREAM // 16):
        ones_v[pl.ds(i * 16, 16)] = jnp.ones((16,), jnp.float32)
    pltpu.sync_copy(zeros_hbm, tmp)
    pltpu.sync_copy(tmp, acc.at[pl.ds(base, RPT)])
    plsc.subcore_barrier()

    def chunk_body(k, carry):
        gb = w * GPW + k * SPC
        pltpu.sync_copy(dst_hbm.at[pl.ds(gb, SPC)], didx)
        for j in range(SPC):
            pltpu.sync_copy(ones_v, acc.at[didx.at[j]], add=True)
        return carry

    lax.fori_loop(0, CPW, chunk_body, 0)
    plsc.subcore_barrier()
    pltpu.sync_copy(acc.at[pl.ds(base, RPT)], tmp)
    pltpu.sync_copy(tmp, out_hbm.at[pl.ds(c * NPAD + base, RPT)])


@functools.partial(
    pl.kernel,
    out_type=jax.ShapeDtypeStruct((NC, NPAD, HID), jnp.float32),
    mesh=_MESH,
    compiler_params=_SC_PARAMS,
    scratch_types=[
        pltpu.VMEM((SPC, STREAM), jnp.int32),
        pltpu.VMEM((SPC, STREAM), jnp.int32),
        pltpu.VMEM((CHUNK, HID), jnp.float32),
        pltpu.VMEM((RPT, HID), jnp.float32),
        pltpu.VMEM_SHARED((NPAD, HID), jnp.float32),
        pltpu.SemaphoreType.DMA,
    ],
)
def _agg_kernel(src_hbm, dst_hbm, feat_hbm, zeros_hbm, out_hbm,
                sidx, didx, rows, tmp, acc, gsem):
    c = lax.axis_index("c")
    s = lax.axis_index("s")
    w = c * NS + s
    base = s * RPT
    pltpu.sync_copy(zeros_hbm, tmp)
    pltpu.sync_copy(tmp, acc.at[pl.ds(base, RPT)])
    plsc.subcore_barrier()

    def chunk_body(k, carry):
        gb = w * GPW + k * SPC
        pltpu.sync_copy(src_hbm.at[pl.ds(gb, SPC)], sidx)
        pltpu.sync_copy(dst_hbm.at[pl.ds(gb, SPC)], didx)
        descs = [
            pltpu.async_copy(feat_hbm.at[sidx.at[j]],
                             rows.at[pl.ds(j * STREAM, STREAM)], gsem)
            for j in range(SPC)
        ]
        for d in descs:
            d.wait()
        for j in range(SPC):
            pltpu.sync_copy(rows.at[pl.ds(j * STREAM, STREAM)],
                            acc.at[didx.at[j]], add=True)
        return carry

    lax.fori_loop(0, CPW, chunk_body, 0)
    plsc.subcore_barrier()
    pltpu.sync_copy(acc.at[pl.ds(base, RPT)], tmp)
    pltpu.sync_copy(tmp, out_hbm.at[c, pl.ds(base, RPT)])


BN = 400
BD = 6400


def _mm_body(x_ref, w_ref, o_ref):
    o_ref[...] = jnp.dot(x_ref[...], w_ref[...],
                         preferred_element_type=jnp.float32)


def _matmul(x, W1):
    return pl.pallas_call(
        _mm_body,
        grid=(N // BN,),
        in_specs=[pl.BlockSpec((BN, IN_F), lambda i: (i, 0)),
                  pl.BlockSpec((IN_F, HID), lambda i: (0, 0))],
        out_specs=pl.BlockSpec((BN, HID), lambda i: (i, 0)),
        out_shape=jax.ShapeDtypeStruct((N, HID), jnp.float32),
    )(x, W1)


def _dinv_body(d_ref, o_ref):
    d = d_ref[...]
    o_ref[...] = lax.rsqrt(d[0] + d[1] + 1.0)


def _dinv(degp):
    return pl.pallas_call(
        _dinv_body,
        grid=(NPAD // BD,),
        in_specs=[pl.BlockSpec((NC, BD, 1), lambda i: (0, i, 0))],
        out_specs=pl.BlockSpec((BD, 1), lambda i: (i, 0)),
        out_shape=jax.ShapeDtypeStruct((NPAD, 1), jnp.float32),
    )(degp)


def _scale_body(h_ref, v_ref, o_ref):
    o_ref[...] = h_ref[...] * v_ref[...]


def _scale(h, dinvc):
    return pl.pallas_call(
        _scale_body,
        grid=(N // BN,),
        in_specs=[pl.BlockSpec((BN, HID), lambda i: (i, 0)),
                  pl.BlockSpec((BN, 1), lambda i: (i, 0))],
        out_specs=pl.BlockSpec((BN, HID), lambda i: (i, 0)),
        out_shape=jax.ShapeDtypeStruct((N, HID), jnp.float32),
    )(h, dinvc)


def _mid_body(p_ref, h_ref, v_ref, b_ref, o_ref):
    v = v_ref[...]
    agg = v * (p_ref[0] + p_ref[1] + h_ref[...])
    g = jnp.maximum(agg + b_ref[...], 0.0)
    o_ref[...] = v * g


def _mid(p1, hs1, dinvc, b1r):
    return pl.pallas_call(
        _mid_body,
        grid=(N // BN,),
        in_specs=[pl.BlockSpec((NC, BN, HID), lambda i: (0, i, 0)),
                  pl.BlockSpec((BN, HID), lambda i: (i, 0)),
                  pl.BlockSpec((BN, 1), lambda i: (i, 0)),
                  pl.BlockSpec((1, HID), lambda i: (0, 0))],
        out_specs=pl.BlockSpec((BN, HID), lambda i: (i, 0)),
        out_shape=jax.ShapeDtypeStruct((N, HID), jnp.float32),
    )(p1, hs1, dinvc, b1r)


def _out_body(p_ref, h_ref, v_ref, w_ref, b_ref, o_ref):
    v = v_ref[...]
    t = v * (p_ref[0] + p_ref[1] + h_ref[...])
    logits = jnp.dot(t, w_ref[...],
                     preferred_element_type=jnp.float32) + b_ref[...]
    col = lax.broadcasted_iota(jnp.int32, logits.shape, 1)
    valid = col < CLS
    neg = jnp.float32(-1e30)
    ml = jnp.max(jnp.where(valid, logits, neg), axis=1, keepdims=True)
    ex = jnp.where(valid, jnp.exp(logits - ml), 0.0)
    lse = jnp.log(jnp.sum(ex, axis=1, keepdims=True))
    o_ref[...] = (logits - ml - lse)[:, :CLS]


def _outk(p2, hs2, dinvc, W2p, b2p):
    return pl.pallas_call(
        _out_body,
        grid=(N // BN,),
        in_specs=[pl.BlockSpec((NC, BN, HID), lambda i: (0, i, 0)),
                  pl.BlockSpec((BN, HID), lambda i: (i, 0)),
                  pl.BlockSpec((BN, 1), lambda i: (i, 0)),
                  pl.BlockSpec((HID, 128), lambda i: (0, 0)),
                  pl.BlockSpec((1, 128), lambda i: (0, 0))],
        out_specs=pl.BlockSpec((BN, CLS), lambda i: (i, 0)),
        out_shape=jax.ShapeDtypeStruct((N, CLS), jnp.float32),
    )(p2, hs2, dinvc, W2p, b2p)


def kernel(x, edge_index, W1, b1, W2, b2):
    src = edge_index[0]
    dst = edge_index[1]
    pad_n = EP - E
    pad_src = (jnp.arange(pad_n, dtype=jnp.int32) * 37) % N
    pad_dst = jnp.full((pad_n,), N, dtype=jnp.int32)
    src_p = jnp.concatenate([src, pad_src]).reshape(GROUPS, STREAM)
    dst_p = jnp.concatenate([dst, pad_dst]).reshape(GROUPS, STREAM)
    zeros1 = jnp.zeros((RPT,), jnp.float32)
    zeros2 = jnp.zeros((RPT, HID), jnp.float32)
    W2p = jnp.zeros((HID, 128), jnp.float32).at[:, :CLS].set(W2)
    b2p = jnp.zeros((1, 128), jnp.float32).at[0, :CLS].set(b2)
    b1r = b1.reshape(1, HID)

    h1 = _matmul(x, W1)
    degp = _deg_kernel(dst_p, zeros1)
    dinvc = _dinv(degp.reshape(NC, NPAD, 1))
    hs1 = _scale(h1, dinvc)
    p1 = _agg_kernel(src_p, dst_p, hs1, zeros2)
    hs2 = _mid(p1, hs1, dinvc, b1r)
    p2 = _agg_kernel(src_p, dst_p, hs2, zeros2)
    return _outk(p2, hs2, dinvc, W2p, b2p)


# speedup vs baseline: 79.8615x; 1.1720x over previous
import functools

import jax
import jax.numpy as jnp
from jax import lax
from jax.experimental import pallas as pl
from jax.experimental.pallas import tpu as pltpu
from jax.experimental.pallas import tpu_sc as plsc

N = 50000
E = 3200000
IN_F = 1433
HID = 16
CLS = 7

NC, NS = 2, 16
NW = NC * NS
RPT = 3200
NPAD = NS * RPT
STREAM = 128
SPC = 8
CHUNK = STREAM * SPC
EP = -(-E // (NW * CHUNK)) * (NW * CHUNK)
GROUPS = EP // STREAM
GPW = GROUPS // NW
CPW = EP // (NW * CHUNK)

_MESH = plsc.VectorSubcoreMesh(core_axis_name="c", subcore_axis_name="s")
_SC_PARAMS = pltpu.CompilerParams(use_tc_tiling_on_sc=False,
                                  internal_scratch_in_bytes=256 * 1024)


@functools.partial(
    pl.kernel,
    out_type=jax.ShapeDtypeStruct((NC * NPAD,), jnp.float32),
    mesh=_MESH,
    compiler_params=_SC_PARAMS,
    scratch_types=[
        pltpu.VMEM((SPC, STREAM), jnp.int32),
        pltpu.VMEM((STREAM,), jnp.float32),
        pltpu.VMEM((RPT,), jnp.float32),
        pltpu.VMEM_SHARED((NPAD,), jnp.float32),
    ],
)
def _deg_kernel(dst_hbm, zeros_hbm, out_hbm, didx, ones_v, tmp, acc):
    c = lax.axis_index("c")
    s = lax.axis_index("s")
    w = c * NS + s
    base = s * RPT
    for i in range(STREAM // 16):
        ones_v[pl.ds(i * 16, 16)] = jnp.ones((16,), jnp.float32)
    pltpu.sync_copy(zeros_hbm, tmp)
    pltpu.sync_copy(tmp, acc.at[pl.ds(base, RPT)])
    plsc.subcore_barrier()

    def chunk_body(k, carry):
        gb = w * GPW + k * SPC
        pltpu.sync_copy(dst_hbm.at[pl.ds(gb, SPC)], didx)
        for j in range(SPC):
            pltpu.sync_copy(ones_v, acc.at[didx.at[j]], add=True)
        return carry

    lax.fori_loop(0, CPW, chunk_body, 0)
    plsc.subcore_barrier()
    pltpu.sync_copy(acc.at[pl.ds(base, RPT)], tmp)
    pltpu.sync_copy(tmp, out_hbm.at[pl.ds(c * NPAD + base, RPT)])


@functools.partial(
    pl.kernel,
    out_type=jax.ShapeDtypeStruct((NC, NPAD, HID), jnp.float32),
    mesh=_MESH,
    compiler_params=_SC_PARAMS,
    scratch_types=[
        pltpu.VMEM((SPC, STREAM), jnp.int32),
        pltpu.VMEM((SPC, STREAM), jnp.int32),
        pltpu.VMEM((CHUNK, HID), jnp.float32),
        pltpu.VMEM((RPT, HID), jnp.float32),
        pltpu.VMEM_SHARED((NPAD, HID), jnp.float32),
        pltpu.SemaphoreType.DMA,
    ],
)
def _agg_kernel(src_hbm, dst_hbm, feat_hbm, zeros_hbm, out_hbm,
                sidx, didx, rows, tmp, acc, gsem):
    c = lax.axis_index("c")
    s = lax.axis_index("s")
    w = c * NS + s
    base = s * RPT
    pltpu.sync_copy(zeros_hbm, tmp)
    pltpu.sync_copy(tmp, acc.at[pl.ds(base, RPT)])
    plsc.subcore_barrier()

    def chunk_body(k, carry):
        gb = w * GPW + k * SPC
        pltpu.sync_copy(src_hbm.at[pl.ds(gb, SPC)], sidx)
        pltpu.sync_copy(dst_hbm.at[pl.ds(gb, SPC)], didx)
        descs = [
            pltpu.async_copy(feat_hbm.at[sidx.at[j]],
                             rows.at[pl.ds(j * STREAM, STREAM)], gsem)
            for j in range(SPC)
        ]
        for d in descs:
            d.wait()
        for j in range(SPC):
            pltpu.sync_copy(rows.at[pl.ds(j * STREAM, STREAM)],
                            acc.at[didx.at[j]], add=True)
        return carry

    lax.fori_loop(0, CPW, chunk_body, 0)
    plsc.subcore_barrier()
    pltpu.sync_copy(acc.at[pl.ds(base, RPT)], tmp)
    pltpu.sync_copy(tmp, out_hbm.at[c, pl.ds(base, RPT)])


BN = 400
BD = 6400


BM = 512


def _mm_body(xt_ref, w_ref, o_ref):
    o_ref[...] = lax.dot_general(xt_ref[...], w_ref[...],
                                 (((0,), (0,)), ((), ())),
                                 preferred_element_type=jnp.float32)


def _matmul(xt, W1):
    return pl.pallas_call(
        _mm_body,
        grid=(-(-N // BM),),
        in_specs=[pl.BlockSpec((IN_F, BM), lambda i: (0, i)),
                  pl.BlockSpec((IN_F, HID), lambda i: (0, 0))],
        out_specs=pl.BlockSpec((BM, HID), lambda i: (i, 0)),
        out_shape=jax.ShapeDtypeStruct((N, HID), jnp.float32),
    )(xt, W1)


def _dinv_body(d_ref, o_ref):
    d = d_ref[...]
    o_ref[...] = lax.rsqrt(d[0] + d[1] + 1.0)


def _dinv(degp):
    return pl.pallas_call(
        _dinv_body,
        grid=(NPAD // BD,),
        in_specs=[pl.BlockSpec((NC, BD, 1), lambda i: (0, i, 0))],
        out_specs=pl.BlockSpec((BD, 1), lambda i: (i, 0)),
        out_shape=jax.ShapeDtypeStruct((NPAD, 1), jnp.float32),
    )(degp)


def _scale_body(h_ref, v_ref, o_ref):
    o_ref[...] = h_ref[...] * v_ref[...]


def _scale(h, dinvc):
    return pl.pallas_call(
        _scale_body,
        grid=(N // BN,),
        in_specs=[pl.BlockSpec((BN, HID), lambda i: (i, 0)),
                  pl.BlockSpec((BN, 1), lambda i: (i, 0))],
        out_specs=pl.BlockSpec((BN, HID), lambda i: (i, 0)),
        out_shape=jax.ShapeDtypeStruct((N, HID), jnp.float32),
    )(h, dinvc)


def _mid_body(p_ref, h_ref, v_ref, b_ref, o_ref):
    v = v_ref[...]
    agg = v * (p_ref[0] + p_ref[1] + h_ref[...])
    g = jnp.maximum(agg + b_ref[...], 0.0)
    o_ref[...] = v * g


def _mid(p1, hs1, dinvc, b1r):
    return pl.pallas_call(
        _mid_body,
        grid=(N // BN,),
        in_specs=[pl.BlockSpec((NC, BN, HID), lambda i: (0, i, 0)),
                  pl.BlockSpec((BN, HID), lambda i: (i, 0)),
                  pl.BlockSpec((BN, 1), lambda i: (i, 0)),
                  pl.BlockSpec((1, HID), lambda i: (0, 0))],
        out_specs=pl.BlockSpec((BN, HID), lambda i: (i, 0)),
        out_shape=jax.ShapeDtypeStruct((N, HID), jnp.float32),
    )(p1, hs1, dinvc, b1r)


def _out_body(p_ref, h_ref, v_ref, w_ref, b_ref, o_ref):
    v = v_ref[...]
    t = v * (p_ref[0] + p_ref[1] + h_ref[...])
    logits = jnp.dot(t, w_ref[...],
                     preferred_element_type=jnp.float32) + b_ref[...]
    col = lax.broadcasted_iota(jnp.int32, logits.shape, 1)
    valid = col < CLS
    neg = jnp.float32(-1e30)
    ml = jnp.max(jnp.where(valid, logits, neg), axis=1, keepdims=True)
    ex = jnp.where(valid, jnp.exp(logits - ml), 0.0)
    lse = jnp.log(jnp.sum(ex, axis=1, keepdims=True))
    o_ref[...] = (logits - ml - lse)[:, :CLS]


def _outk(p2, hs2, dinvc, W2p, b2p):
    return pl.pallas_call(
        _out_body,
        grid=(N // BN,),
        in_specs=[pl.BlockSpec((NC, BN, HID), lambda i: (0, i, 0)),
                  pl.BlockSpec((BN, HID), lambda i: (i, 0)),
                  pl.BlockSpec((BN, 1), lambda i: (i, 0)),
                  pl.BlockSpec((HID, 128), lambda i: (0, 0)),
                  pl.BlockSpec((1, 128), lambda i: (0, 0))],
        out_specs=pl.BlockSpec((BN, CLS), lambda i: (i, 0)),
        out_shape=jax.ShapeDtypeStruct((N, CLS), jnp.float32),
    )(p2, hs2, dinvc, W2p, b2p)


def kernel(x, edge_index, W1, b1, W2, b2):
    src = edge_index[0]
    dst = edge_index[1]
    pad_n = EP - E
    pad_src = (jnp.arange(pad_n, dtype=jnp.int32) * 37) % N
    pad_dst = jnp.full((pad_n,), N, dtype=jnp.int32)
    src_p = jnp.concatenate([src, pad_src]).reshape(GROUPS, STREAM)
    dst_p = jnp.concatenate([dst, pad_dst]).reshape(GROUPS, STREAM)
    zeros1 = jnp.zeros((RPT,), jnp.float32)
    zeros2 = jnp.zeros((RPT, HID), jnp.float32)
    W2p = jnp.zeros((HID, 128), jnp.float32).at[:, :CLS].set(W2)
    b2p = jnp.zeros((1, 128), jnp.float32).at[0, :CLS].set(b2)
    b1r = b1.reshape(1, HID)

    h1 = _matmul(jnp.transpose(x), W1)
    degp = _deg_kernel(dst_p, zeros1)
    dinvc = _dinv(degp.reshape(NC, NPAD, 1))
    hs1 = _scale(h1, dinvc)
    p1 = _agg_kernel(src_p, dst_p, hs1, zeros2)
    hs2 = _mid(p1, hs1, dinvc, b1r)
    p2 = _agg_kernel(src_p, dst_p, hs2, zeros2)
    return _outk(p2, hs2, dinvc, W2p, b2p)


# speedup vs baseline: 120.5754x; 1.5098x over previous
import functools

import jax
import jax.numpy as jnp
from jax import lax
from jax.experimental import pallas as pl
from jax.experimental.pallas import tpu as pltpu
from jax.experimental.pallas import tpu_sc as plsc

N = 50000
E = 3200000
IN_F = 1433
HID = 16
CLS = 7

NC, NS = 2, 16
NW = NC * NS
RPT = 3200
NPAD = NS * RPT
STREAM = 128
SPC = 8
CHUNK = STREAM * SPC
EP = -(-E // (NW * CHUNK)) * (NW * CHUNK)
GROUPS = EP // STREAM
GPW = GROUPS // NW
CPW = EP // (NW * CHUNK)

_MESH = plsc.VectorSubcoreMesh(core_axis_name="c", subcore_axis_name="s")
_SC_PARAMS = pltpu.CompilerParams(use_tc_tiling_on_sc=False,
                                  internal_scratch_in_bytes=256 * 1024)


SGD = 49
NSCD = GPW // SGD
SG = 14
NSC = GPW // SG
ZR = 1600


@functools.partial(
    pl.kernel,
    out_type=jax.ShapeDtypeStruct((NC * NPAD,), jnp.float32),
    mesh=_MESH,
    compiler_params=_SC_PARAMS,
    scratch_types=[
        pltpu.VMEM((SGD, STREAM), jnp.int32),
        pltpu.VMEM((SGD, STREAM), jnp.int32),
        pltpu.VMEM((STREAM,), jnp.float32),
        pltpu.VMEM((RPT,), jnp.float32),
        pltpu.VMEM_SHARED((NPAD,), jnp.float32),
        pltpu.SemaphoreType.DMA,
        pltpu.SemaphoreType.DMA,
        pltpu.SemaphoreType.DMA,
        pltpu.SemaphoreType.DMA,
    ],
)
def _deg_kernel(dst_hbm, zeros_hbm, out_hbm, didx0, didx1, ones_v, tmp, acc,
                isem0, isem1, ssem0, ssem1):
    c = lax.axis_index("c")
    s = lax.axis_index("s")
    w = c * NS + s
    base = s * RPT
    gw = w * GPW
    didx = (didx0, didx1)
    isem = (isem0, isem1)
    ssem = (ssem0, ssem1)

    def fire_i(t, b):
        pltpu.async_copy(dst_hbm.at[pl.ds(gw + t * SGD, SGD)], didx[b],
                         isem[b])

    def wait_i(b):
        pltpu.make_async_copy(dst_hbm.at[pl.ds(0, SGD)], didx[b],
                              isem[b]).wait()

    def fire_s(b):
        def body(j, carry):
            pltpu.async_copy(ones_v, acc.at[didx[b].at[j]], ssem[b],
                             add=True)
            return carry
        lax.fori_loop(0, SGD, body, 0)

    def wait_s(b):
        def body(j, carry):
            pltpu.make_async_copy(ones_v, acc.at[didx[b].at[j]],
                                  ssem[b]).wait()
            return carry
        lax.fori_loop(0, SGD, body, 0)

    for i in range(STREAM // 16):
        ones_v[pl.ds(i * 16, 16)] = jnp.ones((16,), jnp.float32)
    pltpu.sync_copy(zeros_hbm, tmp)
    pltpu.sync_copy(tmp, acc.at[pl.ds(base, RPT)])
    plsc.subcore_barrier()

    fire_i(0, 0)
    wait_i(0)
    fire_s(0)
    fire_i(1, 1)

    def pair(i, carry):
        t0 = 2 * i + 1
        wait_i(1)
        fire_s(1)
        wait_s(0)
        fire_i(t0 + 1, 0)
        wait_i(0)
        fire_s(0)
        wait_s(1)
        fire_i(t0 + 2, 1)
        return carry

    lax.fori_loop(0, (NSCD - 2) // 2, pair, 0)
    wait_i(1)
    fire_s(1)
    wait_s(0)
    wait_s(1)
    plsc.subcore_barrier()
    pltpu.sync_copy(acc.at[pl.ds(base, RPT)], tmp)
    pltpu.sync_copy(tmp, out_hbm.at[pl.ds(c * NPAD + base, RPT)])


@functools.partial(
    pl.kernel,
    out_type=jax.ShapeDtypeStruct((NC, NPAD, HID), jnp.float32),
    mesh=_MESH,
    compiler_params=_SC_PARAMS,
    scratch_types=[
        pltpu.VMEM((SG, STREAM), jnp.int32),
        pltpu.VMEM((SG, STREAM), jnp.int32),
        pltpu.VMEM((SG, STREAM), jnp.int32),
        pltpu.VMEM((SG, STREAM), jnp.int32),
        pltpu.VMEM((SG * STREAM, HID), jnp.float32),
        pltpu.VMEM((SG * STREAM, HID), jnp.float32),
        pltpu.VMEM_SHARED((NPAD, HID), jnp.float32),
        pltpu.SemaphoreType.DMA,
        pltpu.SemaphoreType.DMA,
        pltpu.SemaphoreType.DMA,
        pltpu.SemaphoreType.DMA,
        pltpu.SemaphoreType.DMA,
        pltpu.SemaphoreType.DMA,
    ],
)
def _agg_kernel(src_hbm, dst_hbm, feat_hbm, zeros_hbm, out_hbm,
                sidx0, sidx1, didx0, didx1, rows0, rows1, acc,
                isem0, isem1, gsem0, gsem1, ssem0, ssem1):
    c = lax.axis_index("c")
    s = lax.axis_index("s")
    w = c * NS + s
    base = s * RPT
    gw = w * GPW
    sidx = (sidx0, sidx1)
    didx = (didx0, didx1)
    rows = (rows0, rows1)
    isem = (isem0, isem1)
    gsem = (gsem0, gsem1)
    ssem = (ssem0, ssem1)

    def fire_i(t, b):
        g = gw + t * SG
        pltpu.async_copy(src_hbm.at[pl.ds(g, SG)], sidx[b], isem[b])
        pltpu.async_copy(dst_hbm.at[pl.ds(g, SG)], didx[b], isem[b])

    def wait_i(b):
        pltpu.make_async_copy(src_hbm.at[pl.ds(0, SG)], sidx[b],
                              isem[b]).wait()
        pltpu.make_async_copy(dst_hbm.at[pl.ds(0, SG)], didx[b],
                              isem[b]).wait()

    def fire_g(b):
        def body(j, carry):
            pltpu.async_copy(feat_hbm.at[sidx[b].at[j]],
                             rows[b].at[pl.ds(j * STREAM, STREAM)], gsem[b])
            return carry
        lax.fori_loop(0, SG, body, 0)

    def wait_g(b):
        def body(j, carry):
            pltpu.make_async_copy(feat_hbm.at[sidx[b].at[j]],
                                  rows[b].at[pl.ds(j * STREAM, STREAM)],
                                  gsem[b]).wait()
            return carry
        lax.fori_loop(0, SG, body, 0)

    def fire_s(b):
        def body(j, carry):
            pltpu.async_copy(rows[b].at[pl.ds(j * STREAM, STREAM)],
                             acc.at[didx[b].at[j]], ssem[b], add=True)
            return carry
        lax.fori_loop(0, SG, body, 0)

    def wait_s(b):
        def body(j, carry):
            pltpu.make_async_copy(rows[b].at[pl.ds(j * STREAM, STREAM)],
                                  acc.at[didx[b].at[j]], ssem[b]).wait()
            return carry
        lax.fori_loop(0, SG, body, 0)

    for q in range(RPT // ZR):
        pltpu.sync_copy(zeros_hbm, rows0.at[pl.ds(0, ZR)])
        pltpu.sync_copy(rows0.at[pl.ds(0, ZR)],
                        acc.at[pl.ds(base + q * ZR, ZR)])
    plsc.subcore_barrier()

    fire_i(0, 0)
    wait_i(0)
    fire_g(0)
    fire_i(1, 1)

    def pair(i, carry):
        t0 = 2 * i + 1
        wait_i(1)
        fire_g(1)
        wait_g(0)
        fire_s(0)
        wait_s(0)
        fire_i(t0 + 1, 0)
        wait_i(0)
        fire_g(0)
        wait_g(1)
        fire_s(1)
        wait_s(1)
        fire_i(t0 + 2, 1)
        return carry

    lax.fori_loop(0, (NSC - 2) // 2, pair, 0)
    wait_i(1)
    fire_g(1)
    wait_g(0)
    fire_s(0)
    wait_s(0)
    wait_g(1)
    fire_s(1)
    wait_s(1)
    plsc.subcore_barrier()
    for q in range(RPT // ZR):
        pltpu.sync_copy(acc.at[pl.ds(base + q * ZR, ZR)],
                        rows0.at[pl.ds(0, ZR)])
        pltpu.sync_copy(rows0.at[pl.ds(0, ZR)],
                        out_hbm.at[c, pl.ds(base + q * ZR, ZR)])


BN = 400
BD = 6400


BM = 512


def _mm_body(xt_ref, w_ref, o_ref):
    o_ref[...] = lax.dot_general(xt_ref[...], w_ref[...],
                                 (((0,), (0,)), ((), ())),
                                 preferred_element_type=jnp.float32)


def _matmul(xt, W1):
    return pl.pallas_call(
        _mm_body,
        grid=(-(-N // BM),),
        in_specs=[pl.BlockSpec((IN_F, BM), lambda i: (0, i)),
                  pl.BlockSpec((IN_F, HID), lambda i: (0, 0))],
        out_specs=pl.BlockSpec((BM, HID), lambda i: (i, 0)),
        out_shape=jax.ShapeDtypeStruct((N, HID), jnp.float32),
    )(xt, W1)


def _dinv_body(d_ref, o_ref):
    d = d_ref[...]
    o_ref[...] = lax.rsqrt(d[0] + d[1] + 1.0)


def _dinv(degp):
    return pl.pallas_call(
        _dinv_body,
        grid=(NPAD // BD,),
        in_specs=[pl.BlockSpec((NC, BD, 1), lambda i: (0, i, 0))],
        out_specs=pl.BlockSpec((BD, 1), lambda i: (i, 0)),
        out_shape=jax.ShapeDtypeStruct((NPAD, 1), jnp.float32),
    )(degp)


def _scale_body(h_ref, v_ref, o_ref):
    o_ref[...] = h_ref[...] * v_ref[...]


def _scale(h, dinvc):
    return pl.pallas_call(
        _scale_body,
        grid=(N // BN,),
        in_specs=[pl.BlockSpec((BN, HID), lambda i: (i, 0)),
                  pl.BlockSpec((BN, 1), lambda i: (i, 0))],
        out_specs=pl.BlockSpec((BN, HID), lambda i: (i, 0)),
        out_shape=jax.ShapeDtypeStruct((N, HID), jnp.float32),
    )(h, dinvc)


def _mid_body(p_ref, h_ref, v_ref, b_ref, o_ref):
    v = v_ref[...]
    agg = v * (p_ref[0] + p_ref[1] + h_ref[...])
    g = jnp.maximum(agg + b_ref[...], 0.0)
    o_ref[...] = v * g


def _mid(p1, hs1, dinvc, b1r):
    return pl.pallas_call(
        _mid_body,
        grid=(N // BN,),
        in_specs=[pl.BlockSpec((NC, BN, HID), lambda i: (0, i, 0)),
                  pl.BlockSpec((BN, HID), lambda i: (i, 0)),
                  pl.BlockSpec((BN, 1), lambda i: (i, 0)),
                  pl.BlockSpec((1, HID), lambda i: (0, 0))],
        out_specs=pl.BlockSpec((BN, HID), lambda i: (i, 0)),
        out_shape=jax.ShapeDtypeStruct((N, HID), jnp.float32),
    )(p1, hs1, dinvc, b1r)


def _out_body(p_ref, h_ref, v_ref, w_ref, b_ref, o_ref):
    v = v_ref[...]
    t = v * (p_ref[0] + p_ref[1] + h_ref[...])
    logits = jnp.dot(t, w_ref[...],
                     preferred_element_type=jnp.float32) + b_ref[...]
    col = lax.broadcasted_iota(jnp.int32, logits.shape, 1)
    valid = col < CLS
    neg = jnp.float32(-1e30)
    ml = jnp.max(jnp.where(valid, logits, neg), axis=1, keepdims=True)
    ex = jnp.where(valid, jnp.exp(logits - ml), 0.0)
    lse = jnp.log(jnp.sum(ex, axis=1, keepdims=True))
    o_ref[...] = (logits - ml - lse)[:, :CLS]


def _outk(p2, hs2, dinvc, W2p, b2p):
    return pl.pallas_call(
        _out_body,
        grid=(N // BN,),
        in_specs=[pl.BlockSpec((NC, BN, HID), lambda i: (0, i, 0)),
                  pl.BlockSpec((BN, HID), lambda i: (i, 0)),
                  pl.BlockSpec((BN, 1), lambda i: (i, 0)),
                  pl.BlockSpec((HID, 128), lambda i: (0, 0)),
                  pl.BlockSpec((1, 128), lambda i: (0, 0))],
        out_specs=pl.BlockSpec((BN, CLS), lambda i: (i, 0)),
        out_shape=jax.ShapeDtypeStruct((N, CLS), jnp.float32),
    )(p2, hs2, dinvc, W2p, b2p)


def kernel(x, edge_index, W1, b1, W2, b2):
    src = edge_index[0]
    dst = edge_index[1]
    pad_n = EP - E
    pad_src = (jnp.arange(pad_n, dtype=jnp.int32) * 37) % N
    pad_dst = jnp.full((pad_n,), N, dtype=jnp.int32)
    src_p = jnp.concatenate([src, pad_src]).reshape(GROUPS, STREAM)
    dst_p = jnp.concatenate([dst, pad_dst]).reshape(GROUPS, STREAM)
    zeros1 = jnp.zeros((RPT,), jnp.float32)
    zeros2 = jnp.zeros((ZR, HID), jnp.float32)
    W2p = jnp.zeros((HID, 128), jnp.float32).at[:, :CLS].set(W2)
    b2p = jnp.zeros((1, 128), jnp.float32).at[0, :CLS].set(b2)
    b1r = b1.reshape(1, HID)

    h1 = _matmul(jnp.transpose(x), W1)
    degp = _deg_kernel(dst_p, zeros1)
    dinvc = _dinv(degp.reshape(NC, NPAD, 1))
    hs1 = _scale(h1, dinvc)
    p1 = _agg_kernel(src_p, dst_p, hs1, zeros2)
    hs2 = _mid(p1, hs1, dinvc, b1r)
    p2 = _agg_kernel(src_p, dst_p, hs2, zeros2)
    return _outk(p2, hs2, dinvc, W2p, b2p)


# speedup vs baseline: 146.6882x; 1.2166x over previous
import functools

import jax
import jax.numpy as jnp
from jax import lax
from jax.experimental import pallas as pl
from jax.experimental.pallas import tpu as pltpu
from jax.experimental.pallas import tpu_sc as plsc

N = 50000
E = 3200000
IN_F = 1433
HID = 16
CLS = 7

NC, NS = 2, 16
NW = NC * NS
RPT = 3200
NPAD = NS * RPT
STREAM = 128
SPC = 8
CHUNK = STREAM * SPC
EP = -(-E // (NW * CHUNK)) * (NW * CHUNK)
GROUPS = EP // STREAM
GPW = GROUPS // NW
CPW = EP // (NW * CHUNK)

_MESH = plsc.VectorSubcoreMesh(core_axis_name="c", subcore_axis_name="s")
_SC_PARAMS = pltpu.CompilerParams(use_tc_tiling_on_sc=False,
                                  internal_scratch_in_bytes=256 * 1024)


SGD = 49
NSCD = GPW // SGD
SG = 14
NSC = GPW // SG
ZR = 1600


@functools.partial(
    pl.kernel,
    out_type=jax.ShapeDtypeStruct((NC * NPAD,), jnp.float32),
    mesh=_MESH,
    compiler_params=_SC_PARAMS,
    scratch_types=[
        pltpu.VMEM((SGD, STREAM), jnp.int32),
        pltpu.VMEM((SGD, STREAM), jnp.int32),
        pltpu.VMEM((STREAM,), jnp.float32),
        pltpu.VMEM((RPT,), jnp.float32),
        pltpu.VMEM_SHARED((NPAD,), jnp.float32),
        pltpu.SemaphoreType.DMA,
        pltpu.SemaphoreType.DMA,
        pltpu.SemaphoreType.DMA,
        pltpu.SemaphoreType.DMA,
    ],
)
def _deg_kernel(dst_hbm, zeros_hbm, out_hbm, didx0, didx1, ones_v, tmp, acc,
                isem0, isem1, ssem0, ssem1):
    c = lax.axis_index("c")
    s = lax.axis_index("s")
    w = c * NS + s
    base = s * RPT
    gw = w * GPW
    didx = (didx0, didx1)
    isem = (isem0, isem1)
    ssem = (ssem0, ssem1)

    def fire_i(t, b):
        pltpu.async_copy(dst_hbm.at[pl.ds(gw + t * SGD, SGD)], didx[b],
                         isem[b])

    def wait_i(b):
        pltpu.make_async_copy(dst_hbm.at[pl.ds(0, SGD)], didx[b],
                              isem[b]).wait()

    def fire_s(b):
        def body(j, carry):
            pltpu.async_copy(ones_v, acc.at[didx[b].at[j]], ssem[b],
                             add=True)
            return carry
        lax.fori_loop(0, SGD, body, 0)

    def wait_s(b):
        def body(j, carry):
            pltpu.make_async_copy(ones_v, acc.at[didx[b].at[j]],
                                  ssem[b]).wait()
            return carry
        lax.fori_loop(0, SGD, body, 0)

    for i in range(STREAM // 16):
        ones_v[pl.ds(i * 16, 16)] = jnp.ones((16,), jnp.float32)
    pltpu.sync_copy(zeros_hbm, tmp)
    pltpu.sync_copy(tmp, acc.at[pl.ds(base, RPT)])
    plsc.subcore_barrier()

    fire_i(0, 0)
    wait_i(0)
    fire_s(0)
    fire_i(1, 1)

    def pair(i, carry):
        t0 = 2 * i + 1
        wait_i(1)
        fire_s(1)
        wait_s(0)
        fire_i(t0 + 1, 0)
        wait_i(0)
        fire_s(0)
        wait_s(1)
        fire_i(t0 + 2, 1)
        return carry

    lax.fori_loop(0, (NSCD - 2) // 2, pair, 0)
    wait_i(1)
    fire_s(1)
    wait_s(0)
    wait_s(1)
    plsc.subcore_barrier()
    pltpu.sync_copy(acc.at[pl.ds(base, RPT)], tmp)
    pltpu.sync_copy(tmp, out_hbm.at[pl.ds(c * NPAD + base, RPT)])


@functools.partial(
    pl.kernel,
    out_type=jax.ShapeDtypeStruct((NC, NPAD, HID), jnp.float32),
    mesh=_MESH,
    compiler_params=_SC_PARAMS,
    scratch_types=[
        pltpu.VMEM((SG, STREAM), jnp.int32),
        pltpu.VMEM((SG, STREAM), jnp.int32),
        pltpu.VMEM((SG, STREAM), jnp.int32),
        pltpu.VMEM((SG, STREAM), jnp.int32),
        pltpu.VMEM((SG * STREAM, HID), jnp.float32),
        pltpu.VMEM((SG * STREAM, HID), jnp.float32),
        pltpu.VMEM_SHARED((NPAD, HID), jnp.float32),
        pltpu.SemaphoreType.DMA,
        pltpu.SemaphoreType.DMA,
        pltpu.SemaphoreType.DMA,
        pltpu.SemaphoreType.DMA,
        pltpu.SemaphoreType.DMA,
        pltpu.SemaphoreType.DMA,
    ],
)
def _agg_kernel(src_hbm, dst_hbm, feat_hbm, zeros_hbm, out_hbm,
                sidx0, sidx1, didx0, didx1, rows0, rows1, acc,
                isem0, isem1, gsem0, gsem1, ssem0, ssem1):
    c = lax.axis_index("c")
    s = lax.axis_index("s")
    w = c * NS + s
    base = s * RPT
    gw = w * GPW
    sidx = (sidx0, sidx1)
    didx = (didx0, didx1)
    rows = (rows0, rows1)
    isem = (isem0, isem1)
    gsem = (gsem0, gsem1)
    ssem = (ssem0, ssem1)

    def fire_i(t, b):
        g = gw + t * SG
        pltpu.async_copy(src_hbm.at[pl.ds(g, SG)], sidx[b], isem[b])
        pltpu.async_copy(dst_hbm.at[pl.ds(g, SG)], didx[b], isem[b])

    def wait_i(b):
        pltpu.make_async_copy(src_hbm.at[pl.ds(0, SG)], sidx[b],
                              isem[b]).wait()
        pltpu.make_async_copy(dst_hbm.at[pl.ds(0, SG)], didx[b],
                              isem[b]).wait()

    def fire_g(b):
        def body(j, carry):
            pltpu.async_copy(feat_hbm.at[sidx[b].at[j]],
                             rows[b].at[pl.ds(j * STREAM, STREAM)], gsem[b])
            return carry
        lax.fori_loop(0, SG, body, 0)

    def wait_g(b):
        def body(j, carry):
            pltpu.make_async_copy(feat_hbm.at[sidx[b].at[j]],
                                  rows[b].at[pl.ds(j * STREAM, STREAM)],
                                  gsem[b]).wait()
            return carry
        lax.fori_loop(0, SG, body, 0)

    def fire_s(b):
        def body(j, carry):
            pltpu.async_copy(rows[b].at[pl.ds(j * STREAM, STREAM)],
                             acc.at[didx[b].at[j]], ssem[b], add=True)
            return carry
        lax.fori_loop(0, SG, body, 0)

    def wait_s(b):
        def body(j, carry):
            pltpu.make_async_copy(rows[b].at[pl.ds(j * STREAM, STREAM)],
                                  acc.at[didx[b].at[j]], ssem[b]).wait()
            return carry
        lax.fori_loop(0, SG, body, 0)

    for q in range(RPT // ZR):
        pltpu.sync_copy(zeros_hbm, rows0.at[pl.ds(0, ZR)])
        pltpu.sync_copy(rows0.at[pl.ds(0, ZR)],
                        acc.at[pl.ds(base + q * ZR, ZR)])
    plsc.subcore_barrier()

    fire_i(0, 0)
    wait_i(0)
    fire_g(0)
    fire_i(1, 1)

    def pair(i, carry):
        t0 = 2 * i + 1
        wait_i(1)
        fire_g(1)
        wait_g(0)
        fire_s(0)
        wait_s(0)
        fire_i(t0 + 1, 0)
        wait_i(0)
        fire_g(0)
        wait_g(1)
        fire_s(1)
        wait_s(1)
        fire_i(t0 + 2, 1)
        return carry

    lax.fori_loop(0, (NSC - 2) // 2, pair, 0)
    wait_i(1)
    fire_g(1)
    wait_g(0)
    fire_s(0)
    wait_s(0)
    wait_g(1)
    fire_s(1)
    wait_s(1)
    plsc.subcore_barrier()
    for q in range(RPT // ZR):
        pltpu.sync_copy(acc.at[pl.ds(base + q * ZR, ZR)],
                        rows0.at[pl.ds(0, ZR)])
        pltpu.sync_copy(rows0.at[pl.ds(0, ZR)],
                        out_hbm.at[c, pl.ds(base + q * ZR, ZR)])


BN = 400
BD = 5120


BM = 512


def _mm_body(w_ref, xt_ref, o_ref):
    o_ref[...] = lax.dot_general(w_ref[...], xt_ref[...],
                                 (((0,), (0,)), ((), ())),
                                 preferred_element_type=jnp.float32)


def _matmul(xt, W1):
    return pl.pallas_call(
        _mm_body,
        grid=(-(-N // BM),),
        in_specs=[pl.BlockSpec((IN_F, HID), lambda i: (0, 0)),
                  pl.BlockSpec((IN_F, BM), lambda i: (0, i))],
        out_specs=pl.BlockSpec((HID, BM), lambda i: (0, i)),
        out_shape=jax.ShapeDtypeStruct((HID, N), jnp.float32),
    )(W1, xt)


def _dinv_body(d0_ref, d1_ref, o_ref):
    o_ref[...] = lax.rsqrt(d0_ref[...] + d1_ref[...] + 1.0)[None, :]


def _dinv(degp):
    nb = NPAD // BD
    return pl.pallas_call(
        _dinv_body,
        grid=(nb,),
        in_specs=[pl.BlockSpec((BD,), lambda i: (i,)),
                  pl.BlockSpec((BD,), lambda i, _nb=nb: (i + _nb,))],
        out_specs=pl.BlockSpec((1, BD), lambda i: (0, i)),
        out_shape=jax.ShapeDtypeStruct((1, NPAD), jnp.float32),
    )(degp, degp)


def _scale_body(h_ref, v_ref, o_ref):
    o_ref[...] = h_ref[...] * v_ref[...]


def _scale(ht, dinvr):
    return pl.pallas_call(
        _scale_body,
        grid=(-(-N // BM),),
        in_specs=[pl.BlockSpec((HID, BM), lambda i: (0, i)),
                  pl.BlockSpec((1, BM), lambda i: (0, i))],
        out_specs=pl.BlockSpec((HID, BM), lambda i: (0, i)),
        out_shape=jax.ShapeDtypeStruct((HID, N), jnp.float32),
    )(ht, dinvr)


def _mid_body(p_ref, h_ref, v_ref, b_ref, o_ref):
    v = v_ref[...]
    agg = v * (p_ref[0] + p_ref[1] + h_ref[...])
    g = jnp.maximum(agg + b_ref[...], 0.0)
    o_ref[...] = v * g


def _mid(p1t, hs1t, dinvr, b1c):
    return pl.pallas_call(
        _mid_body,
        grid=(-(-N // BM),),
        in_specs=[pl.BlockSpec((NC, HID, BM), lambda i: (0, 0, i)),
                  pl.BlockSpec((HID, BM), lambda i: (0, i)),
                  pl.BlockSpec((1, BM), lambda i: (0, i)),
                  pl.BlockSpec((HID, 1), lambda i: (0, 0))],
        out_specs=pl.BlockSpec((HID, BM), lambda i: (0, i)),
        out_shape=jax.ShapeDtypeStruct((HID, N), jnp.float32),
    )(p1t, hs1t, dinvr, b1c)


def _out_body(p_ref, h_ref, v_ref, w_ref, b_ref, o_ref):
    v = v_ref[...]
    t = v * (p_ref[0] + p_ref[1] + h_ref[...])
    logits = lax.dot_general(w_ref[...], t, (((0,), (0,)), ((), ())),
                             preferred_element_type=jnp.float32) + b_ref[...]
    ml = jnp.max(logits, axis=0, keepdims=True)
    lse = jnp.log(jnp.sum(jnp.exp(logits - ml), axis=0, keepdims=True))
    o_ref[...] = logits - ml - lse


def _outk(p2t, hs2t, dinvr, W2, b2c):
    return pl.pallas_call(
        _out_body,
        grid=(-(-N // BM),),
        in_specs=[pl.BlockSpec((NC, HID, BM), lambda i: (0, 0, i)),
                  pl.BlockSpec((HID, BM), lambda i: (0, i)),
                  pl.BlockSpec((1, BM), lambda i: (0, i)),
                  pl.BlockSpec((HID, CLS), lambda i: (0, 0)),
                  pl.BlockSpec((CLS, 1), lambda i: (0, 0))],
        out_specs=pl.BlockSpec((CLS, BM), lambda i: (0, i)),
        out_shape=jax.ShapeDtypeStruct((CLS, N), jnp.float32),
    )(p2t, hs2t, dinvr, W2, b2c)


def kernel(x, edge_index, W1, b1, W2, b2):
    src = edge_index[0]
    dst = edge_index[1]
    pad_n = EP - E
    pad_src = (jnp.arange(pad_n, dtype=jnp.int32) * 37) % N
    pad_dst = jnp.full((pad_n,), N, dtype=jnp.int32)
    src_p = jnp.concatenate([src, pad_src]).reshape(GROUPS, STREAM)
    dst_p = jnp.concatenate([dst, pad_dst]).reshape(GROUPS, STREAM)
    zeros1 = jnp.zeros((RPT,), jnp.float32)
    zeros2 = jnp.zeros((ZR, HID), jnp.float32)
    b1c = b1.reshape(HID, 1)
    b2c = b2.reshape(CLS, 1)

    h1t = _matmul(jnp.transpose(x), W1)
    degp = _deg_kernel(dst_p, zeros1)
    dinvr = _dinv(degp)
    hs1t = _scale(h1t, dinvr)
    hs1 = jnp.transpose(hs1t)
    p1 = _agg_kernel(src_p, dst_p, hs1, zeros2)
    hs2t = _mid(jnp.transpose(p1, (0, 2, 1)), hs1t, dinvr, b1c)
    hs2 = jnp.transpose(hs2t)
    p2 = _agg_kernel(src_p, dst_p, hs2, zeros2)
    outt = _outk(jnp.transpose(p2, (0, 2, 1)), hs2t, dinvr, W2, b2c)
    return jnp.transpose(outt)


# speedup vs baseline: 146.7696x; 1.0006x over previous
import functools

import jax
import jax.numpy as jnp
from jax import lax
from jax.experimental import pallas as pl
from jax.experimental.pallas import tpu as pltpu
from jax.experimental.pallas import tpu_sc as plsc

N = 50000
E = 3200000
IN_F = 1433
HID = 16
CLS = 7

NC, NS = 2, 16
NW = NC * NS
RPT = 3200
NPAD = NS * RPT
STREAM = 128
SPC = 8
CHUNK = STREAM * SPC
EP = -(-E // (NW * CHUNK)) * (NW * CHUNK)
GROUPS = EP // STREAM
GPW = GROUPS // NW
CPW = EP // (NW * CHUNK)

_MESH = plsc.VectorSubcoreMesh(core_axis_name="c", subcore_axis_name="s")
_SC_PARAMS = pltpu.CompilerParams(use_tc_tiling_on_sc=False,
                                  internal_scratch_in_bytes=256 * 1024)


SGD = 49
NSCD = GPW // SGD
SG = 14
NSC = GPW // SG
ZR = 1600


@functools.partial(
    pl.kernel,
    out_type=jax.ShapeDtypeStruct((NC * NPAD,), jnp.float32),
    mesh=_MESH,
    compiler_params=_SC_PARAMS,
    scratch_types=[
        pltpu.VMEM((SGD, STREAM), jnp.int32),
        pltpu.VMEM((SGD, STREAM), jnp.int32),
        pltpu.VMEM((STREAM,), jnp.float32),
        pltpu.VMEM((RPT,), jnp.float32),
        pltpu.VMEM_SHARED((NPAD,), jnp.float32),
        pltpu.SemaphoreType.DMA,
        pltpu.SemaphoreType.DMA,
        pltpu.SemaphoreType.DMA,
        pltpu.SemaphoreType.DMA,
    ],
)
def _deg_kernel(dst_hbm, zeros_hbm, out_hbm, didx0, didx1, ones_v, tmp, acc,
                isem0, isem1, ssem0, ssem1):
    c = lax.axis_index("c")
    s = lax.axis_index("s")
    w = c * NS + s
    base = s * RPT
    gw = w * GPW
    didx = (didx0, didx1)
    isem = (isem0, isem1)
    ssem = (ssem0, ssem1)

    def fire_i(t, b):
        pltpu.async_copy(dst_hbm.at[pl.ds(gw + t * SGD, SGD)], didx[b],
                         isem[b])

    def wait_i(b):
        pltpu.make_async_copy(dst_hbm.at[pl.ds(0, SGD)], didx[b],
                              isem[b]).wait()

    def fire_s(b):
        def body(j, carry):
            pltpu.async_copy(ones_v, acc.at[didx[b].at[j]], ssem[b],
                             add=True)
            return carry
        lax.fori_loop(0, SGD, body, 0)

    def wait_s(b):
        def body(j, carry):
            pltpu.make_async_copy(ones_v, acc.at[didx[b].at[j]],
                                  ssem[b]).wait()
            return carry
        lax.fori_loop(0, SGD, body, 0)

    for i in range(STREAM // 16):
        ones_v[pl.ds(i * 16, 16)] = jnp.ones((16,), jnp.float32)
    pltpu.sync_copy(zeros_hbm, tmp)
    pltpu.sync_copy(tmp, acc.at[pl.ds(base, RPT)])
    plsc.subcore_barrier()

    fire_i(0, 0)
    wait_i(0)
    fire_s(0)
    fire_i(1, 1)

    def pair(i, carry):
        t0 = 2 * i + 1
        wait_i(1)
        fire_s(1)
        wait_s(0)
        fire_i(t0 + 1, 0)
        wait_i(0)
        fire_s(0)
        wait_s(1)
        fire_i(t0 + 2, 1)
        return carry

    lax.fori_loop(0, (NSCD - 2) // 2, pair, 0)
    wait_i(1)
    fire_s(1)
    wait_s(0)
    wait_s(1)
    plsc.subcore_barrier()
    pltpu.sync_copy(acc.at[pl.ds(base, RPT)], tmp)
    pltpu.sync_copy(tmp, out_hbm.at[pl.ds(c * NPAD + base, RPT)])


@functools.partial(
    pl.kernel,
    out_type=jax.ShapeDtypeStruct((NC, NPAD, HID), jnp.float32),
    mesh=_MESH,
    compiler_params=_SC_PARAMS,
    scratch_types=[
        pltpu.VMEM((SG, STREAM), jnp.int32),
        pltpu.VMEM((SG, STREAM), jnp.int32),
        pltpu.VMEM((SG, STREAM), jnp.int32),
        pltpu.VMEM((SG, STREAM), jnp.int32),
        pltpu.VMEM((SG * STREAM, HID), jnp.float32),
        pltpu.VMEM((SG * STREAM, HID), jnp.float32),
        pltpu.VMEM_SHARED((NPAD, HID), jnp.float32),
        pltpu.SemaphoreType.DMA,
        pltpu.SemaphoreType.DMA,
        pltpu.SemaphoreType.DMA,
        pltpu.SemaphoreType.DMA,
        pltpu.SemaphoreType.DMA,
        pltpu.SemaphoreType.DMA,
    ],
)
def _agg_kernel(src_hbm, dst_hbm, feat_hbm, zeros_hbm, out_hbm,
                sidx0, sidx1, didx0, didx1, rows0, rows1, acc,
                isem0, isem1, gsem0, gsem1, ssem0, ssem1):
    c = lax.axis_index("c")
    s = lax.axis_index("s")
    w = c * NS + s
    base = s * RPT
    gw = w * GPW
    sidx = (sidx0, sidx1)
    didx = (didx0, didx1)
    rows = (rows0, rows1)
    isem = (isem0, isem1)
    gsem = (gsem0, gsem1)
    ssem = (ssem0, ssem1)

    def fire_i(t, b):
        g = gw + t * SG
        pltpu.async_copy(src_hbm.at[pl.ds(g, SG)], sidx[b], isem[b])
        pltpu.async_copy(dst_hbm.at[pl.ds(g, SG)], didx[b], isem[b])

    def wait_i(b):
        pltpu.make_async_copy(src_hbm.at[pl.ds(0, SG)], sidx[b],
                              isem[b]).wait()
        pltpu.make_async_copy(dst_hbm.at[pl.ds(0, SG)], didx[b],
                              isem[b]).wait()

    def fire_g(b):
        def body(j, carry):
            pltpu.async_copy(feat_hbm.at[sidx[b].at[j]],
                             rows[b].at[pl.ds(j * STREAM, STREAM)], gsem[b])
            return carry
        lax.fori_loop(0, SG, body, 0)

    def wait_g(b):
        def body(j, carry):
            pltpu.make_async_copy(feat_hbm.at[sidx[b].at[j]],
                                  rows[b].at[pl.ds(j * STREAM, STREAM)],
                                  gsem[b]).wait()
            return carry
        lax.fori_loop(0, SG, body, 0)

    def fire_s(b):
        def body(j, carry):
            pltpu.async_copy(rows[b].at[pl.ds(j * STREAM, STREAM)],
                             acc.at[didx[b].at[j]], ssem[b], add=True)
            return carry
        lax.fori_loop(0, SG, body, 0)

    def wait_s(b):
        def body(j, carry):
            pltpu.make_async_copy(rows[b].at[pl.ds(j * STREAM, STREAM)],
                                  acc.at[didx[b].at[j]], ssem[b]).wait()
            return carry
        lax.fori_loop(0, SG, body, 0)

    for q in range(RPT // ZR):
        pltpu.sync_copy(zeros_hbm, rows0.at[pl.ds(0, ZR)])
        pltpu.sync_copy(rows0.at[pl.ds(0, ZR)],
                        acc.at[pl.ds(base + q * ZR, ZR)])
    plsc.subcore_barrier()

    fire_i(0, 0)
    wait_i(0)
    fire_g(0)
    fire_i(1, 1)

    def pair(i, carry):
        t0 = 2 * i + 1
        wait_i(1)
        fire_g(1)
        wait_g(0)
        fire_s(0)
        wait_s(0)
        fire_i(t0 + 1, 0)
        wait_i(0)
        fire_g(0)
        wait_g(1)
        fire_s(1)
        wait_s(1)
        fire_i(t0 + 2, 1)
        return carry

    lax.fori_loop(0, (NSC - 2) // 2, pair, 0)
    wait_i(1)
    fire_g(1)
    wait_g(0)
    fire_s(0)
    wait_s(0)
    wait_g(1)
    fire_s(1)
    wait_s(1)
    plsc.subcore_barrier()
    for q in range(RPT // ZR):
        pltpu.sync_copy(acc.at[pl.ds(base + q * ZR, ZR)],
                        rows0.at[pl.ds(0, ZR)])
        pltpu.sync_copy(rows0.at[pl.ds(0, ZR)],
                        out_hbm.at[c, pl.ds(base + q * ZR, ZR)])


BN = 400
BD = 5120


BM = 512


def _mm_body(w_ref, xt_ref, o_ref):
    o_ref[...] = lax.dot_general(w_ref[...].astype(jnp.bfloat16),
                                 xt_ref[...].astype(jnp.bfloat16),
                                 (((0,), (0,)), ((), ())),
                                 preferred_element_type=jnp.float32)


def _matmul(xt, W1):
    return pl.pallas_call(
        _mm_body,
        grid=(-(-N // BM),),
        in_specs=[pl.BlockSpec((IN_F, HID), lambda i: (0, 0)),
                  pl.BlockSpec((IN_F, BM), lambda i: (0, i))],
        out_specs=pl.BlockSpec((HID, BM), lambda i: (0, i)),
        out_shape=jax.ShapeDtypeStruct((HID, N), jnp.float32),
    )(W1, xt)


def _dinv_body(d0_ref, d1_ref, o_ref):
    o_ref[...] = lax.rsqrt(d0_ref[...] + d1_ref[...] + 1.0)[None, :]


def _dinv(degp):
    nb = NPAD // BD
    return pl.pallas_call(
        _dinv_body,
        grid=(nb,),
        in_specs=[pl.BlockSpec((BD,), lambda i: (i,)),
                  pl.BlockSpec((BD,), lambda i, _nb=nb: (i + _nb,))],
        out_specs=pl.BlockSpec((1, BD), lambda i: (0, i)),
        out_shape=jax.ShapeDtypeStruct((1, NPAD), jnp.float32),
    )(degp, degp)


def _scale_body(h_ref, v_ref, o_ref):
    o_ref[...] = h_ref[...] * v_ref[...]


def _scale(ht, dinvr):
    return pl.pallas_call(
        _scale_body,
        grid=(-(-N // BM),),
        in_specs=[pl.BlockSpec((HID, BM), lambda i: (0, i)),
                  pl.BlockSpec((1, BM), lambda i: (0, i))],
        out_specs=pl.BlockSpec((HID, BM), lambda i: (0, i)),
        out_shape=jax.ShapeDtypeStruct((HID, N), jnp.float32),
    )(ht, dinvr)


def _mid_body(p_ref, h_ref, v_ref, b_ref, o_ref):
    v = v_ref[...]
    agg = v * (p_ref[0] + p_ref[1] + h_ref[...])
    g = jnp.maximum(agg + b_ref[...], 0.0)
    o_ref[...] = v * g


def _mid(p1t, hs1t, dinvr, b1c):
    return pl.pallas_call(
        _mid_body,
        grid=(-(-N // BM),),
        in_specs=[pl.BlockSpec((NC, HID, BM), lambda i: (0, 0, i)),
                  pl.BlockSpec((HID, BM), lambda i: (0, i)),
                  pl.BlockSpec((1, BM), lambda i: (0, i)),
                  pl.BlockSpec((HID, 1), lambda i: (0, 0))],
        out_specs=pl.BlockSpec((HID, BM), lambda i: (0, i)),
        out_shape=jax.ShapeDtypeStruct((HID, N), jnp.float32),
    )(p1t, hs1t, dinvr, b1c)


def _out_body(p_ref, h_ref, v_ref, w_ref, b_ref, o_ref):
    v = v_ref[...]
    t = v * (p_ref[0] + p_ref[1] + h_ref[...])
    logits = lax.dot_general(w_ref[...], t, (((0,), (0,)), ((), ())),
                             preferred_element_type=jnp.float32) + b_ref[...]
    ml = jnp.max(logits, axis=0, keepdims=True)
    lse = jnp.log(jnp.sum(jnp.exp(logits - ml), axis=0, keepdims=True))
    o_ref[...] = logits - ml - lse


def _outk(p2t, hs2t, dinvr, W2, b2c):
    return pl.pallas_call(
        _out_body,
        grid=(-(-N // BM),),
        in_specs=[pl.BlockSpec((NC, HID, BM), lambda i: (0, 0, i)),
                  pl.BlockSpec((HID, BM), lambda i: (0, i)),
                  pl.BlockSpec((1, BM), lambda i: (0, i)),
                  pl.BlockSpec((HID, CLS), lambda i: (0, 0)),
                  pl.BlockSpec((CLS, 1), lambda i: (0, 0))],
        out_specs=pl.BlockSpec((CLS, BM), lambda i: (0, i)),
        out_shape=jax.ShapeDtypeStruct((CLS, N), jnp.float32),
    )(p2t, hs2t, dinvr, W2, b2c)


def kernel(x, edge_index, W1, b1, W2, b2):
    src = edge_index[0]
    dst = edge_index[1]
    pad_n = EP - E
    pad_src = (jnp.arange(pad_n, dtype=jnp.int32) * 37) % N
    pad_dst = jnp.full((pad_n,), N, dtype=jnp.int32)
    src_p = jnp.concatenate([src, pad_src]).reshape(GROUPS, STREAM)
    dst_p = jnp.concatenate([dst, pad_dst]).reshape(GROUPS, STREAM)
    zeros1 = jnp.zeros((RPT,), jnp.float32)
    zeros2 = jnp.zeros((ZR, HID), jnp.float32)
    b1c = b1.reshape(HID, 1)
    b2c = b2.reshape(CLS, 1)

    h1t = _matmul(jnp.transpose(x), W1)
    degp = _deg_kernel(dst_p, zeros1)
    dinvr = _dinv(degp)
    hs1t = _scale(h1t, dinvr)
    hs1 = jnp.transpose(hs1t)
    p1 = _agg_kernel(src_p, dst_p, hs1, zeros2)
    hs2t = _mid(jnp.transpose(p1, (0, 2, 1)), hs1t, dinvr, b1c)
    hs2 = jnp.transpose(hs2t)
    p2 = _agg_kernel(src_p, dst_p, hs2, zeros2)
    outt = _outk(jnp.transpose(p2, (0, 2, 1)), hs2t, dinvr, W2, b2c)
    return jnp.transpose(outt)


# speedup vs baseline: 166.9063x; 1.1372x over previous
import functools

import jax
import jax.numpy as jnp
from jax import lax
from jax.experimental import pallas as pl
from jax.experimental.pallas import tpu as pltpu
from jax.experimental.pallas import tpu_sc as plsc

N = 50000
E = 3200000
IN_F = 1433
HID = 16
CLS = 7

NC, NS = 2, 16
NW = NC * NS
RPT = 3200
NPAD = NS * RPT
STREAM = 128
SPC = 8
CHUNK = STREAM * SPC
EP = -(-E // (NW * CHUNK)) * (NW * CHUNK)
GROUPS = EP // STREAM
GPW = GROUPS // NW
CPW = EP // (NW * CHUNK)

_MESH = plsc.VectorSubcoreMesh(core_axis_name="c", subcore_axis_name="s")
_SC_PARAMS = pltpu.CompilerParams(use_tc_tiling_on_sc=False,
                                  internal_scratch_in_bytes=256 * 1024)


SGD = 49
NSCD = GPW // SGD
SG = 14
NSC = GPW // SG
ZR = 1600


@functools.partial(
    pl.kernel,
    out_type=jax.ShapeDtypeStruct((NC * NPAD,), jnp.float32),
    mesh=_MESH,
    compiler_params=_SC_PARAMS,
    scratch_types=[
        pltpu.VMEM((SGD, STREAM), jnp.int32),
        pltpu.VMEM((SGD, STREAM), jnp.int32),
        pltpu.VMEM((STREAM,), jnp.float32),
        pltpu.VMEM((RPT,), jnp.float32),
        pltpu.VMEM_SHARED((NPAD,), jnp.float32),
        pltpu.SemaphoreType.DMA,
        pltpu.SemaphoreType.DMA,
        pltpu.SemaphoreType.DMA,
        pltpu.SemaphoreType.DMA,
    ],
)
def _deg_kernel(dst_hbm, zeros_hbm, out_hbm, didx0, didx1, ones_v, tmp, acc,
                isem0, isem1, ssem0, ssem1):
    c = lax.axis_index("c")
    s = lax.axis_index("s")
    w = c * NS + s
    base = s * RPT
    gw = w * GPW
    didx = (didx0, didx1)
    isem = (isem0, isem1)
    ssem = (ssem0, ssem1)

    def fire_i(t, b):
        pltpu.async_copy(dst_hbm.at[pl.ds(gw + t * SGD, SGD)], didx[b],
                         isem[b])

    def wait_i(b):
        pltpu.make_async_copy(dst_hbm.at[pl.ds(0, SGD)], didx[b],
                              isem[b]).wait()

    def fire_s(b):
        def body(j, carry):
            pltpu.async_copy(ones_v, acc.at[didx[b].at[j]], ssem[b],
                             add=True)
            return carry
        lax.fori_loop(0, SGD, body, 0)

    def wait_s(b):
        def body(j, carry):
            pltpu.make_async_copy(ones_v, acc.at[didx[b].at[j]],
                                  ssem[b]).wait()
            return carry
        lax.fori_loop(0, SGD, body, 0)

    for i in range(STREAM // 16):
        ones_v[pl.ds(i * 16, 16)] = jnp.ones((16,), jnp.float32)
    pltpu.sync_copy(zeros_hbm, tmp)
    pltpu.sync_copy(tmp, acc.at[pl.ds(base, RPT)])
    plsc.subcore_barrier()

    fire_i(0, 0)
    wait_i(0)
    fire_s(0)
    fire_i(1, 1)

    def pair(i, carry):
        t0 = 2 * i + 1
        wait_i(1)
        fire_s(1)
        wait_s(0)
        fire_i(t0 + 1, 0)
        wait_i(0)
        fire_s(0)
        wait_s(1)
        fire_i(t0 + 2, 1)
        return carry

    lax.fori_loop(0, (NSCD - 2) // 2, pair, 0)
    wait_i(1)
    fire_s(1)
    wait_s(0)
    wait_s(1)
    plsc.subcore_barrier()
    pltpu.sync_copy(acc.at[pl.ds(base, RPT)], tmp)
    pltpu.sync_copy(tmp, out_hbm.at[pl.ds(c * NPAD + base, RPT)])


@functools.partial(
    pl.kernel,
    out_type=jax.ShapeDtypeStruct((NC, NPAD, HID), jnp.float32),
    mesh=_MESH,
    compiler_params=_SC_PARAMS,
    scratch_types=[
        pltpu.VMEM((SG, STREAM), jnp.int32),
        pltpu.VMEM((SG, STREAM), jnp.int32),
        pltpu.VMEM((SG, STREAM), jnp.int32),
        pltpu.VMEM((SG, STREAM), jnp.int32),
        pltpu.VMEM((SG * STREAM, HID), jnp.float32),
        pltpu.VMEM((SG * STREAM, HID), jnp.float32),
        pltpu.VMEM_SHARED((NPAD, HID), jnp.float32),
        pltpu.SemaphoreType.DMA,
        pltpu.SemaphoreType.DMA,
        pltpu.SemaphoreType.DMA,
        pltpu.SemaphoreType.DMA,
        pltpu.SemaphoreType.DMA,
        pltpu.SemaphoreType.DMA,
    ],
)
def _agg_kernel(src_hbm, dst_hbm, feat_hbm, zeros_hbm, out_hbm,
                sidx0, sidx1, didx0, didx1, rows0, rows1, acc,
                isem0, isem1, gsem0, gsem1, ssem0, ssem1):
    c = lax.axis_index("c")
    s = lax.axis_index("s")
    w = c * NS + s
    base = s * RPT
    gw = w * GPW
    sidx = (sidx0, sidx1)
    didx = (didx0, didx1)
    rows = (rows0, rows1)
    isem = (isem0, isem1)
    gsem = (gsem0, gsem1)
    ssem = (ssem0, ssem1)

    def fire_i(t, b):
        g = gw + t * SG
        pltpu.async_copy(src_hbm.at[pl.ds(g, SG)], sidx[b], isem[b])
        pltpu.async_copy(dst_hbm.at[pl.ds(g, SG)], didx[b], isem[b])

    def wait_i(b):
        pltpu.make_async_copy(src_hbm.at[pl.ds(0, SG)], sidx[b],
                              isem[b]).wait()
        pltpu.make_async_copy(dst_hbm.at[pl.ds(0, SG)], didx[b],
                              isem[b]).wait()

    def fire_g(b):
        def body(j, carry):
            pltpu.async_copy(feat_hbm.at[sidx[b].at[j]],
                             rows[b].at[pl.ds(j * STREAM, STREAM)], gsem[b])
            return carry
        lax.fori_loop(0, SG, body, 0)

    def wait_g(b):
        def body(j, carry):
            pltpu.make_async_copy(feat_hbm.at[sidx[b].at[j]],
                                  rows[b].at[pl.ds(j * STREAM, STREAM)],
                                  gsem[b]).wait()
            return carry
        lax.fori_loop(0, SG, body, 0)

    def fire_s(b):
        def body(j, carry):
            pltpu.async_copy(rows[b].at[pl.ds(j * STREAM, STREAM)],
                             acc.at[didx[b].at[j]], ssem[b], add=True)
            return carry
        lax.fori_loop(0, SG, body, 0)

    def wait_s(b):
        def body(j, carry):
            pltpu.make_async_copy(rows[b].at[pl.ds(j * STREAM, STREAM)],
                                  acc.at[didx[b].at[j]], ssem[b]).wait()
            return carry
        lax.fori_loop(0, SG, body, 0)

    for q in range(RPT // ZR):
        pltpu.sync_copy(zeros_hbm, rows0.at[pl.ds(0, ZR)])
        pltpu.sync_copy(rows0.at[pl.ds(0, ZR)],
                        acc.at[pl.ds(base + q * ZR, ZR)])
    plsc.subcore_barrier()

    fire_i(0, 0)
    wait_i(0)
    fire_g(0)
    fire_i(1, 1)

    def pair(i, carry):
        t0 = 2 * i + 1
        wait_i(1)
        fire_g(1)
        wait_g(0)
        fire_s(0)
        wait_s(0)
        fire_i(t0 + 1, 0)
        wait_i(0)
        fire_g(0)
        wait_g(1)
        fire_s(1)
        wait_s(1)
        fire_i(t0 + 2, 1)
        return carry

    lax.fori_loop(0, (NSC - 2) // 2, pair, 0)
    wait_i(1)
    fire_g(1)
    wait_g(0)
    fire_s(0)
    wait_s(0)
    wait_g(1)
    fire_s(1)
    wait_s(1)
    plsc.subcore_barrier()
    for q in range(RPT // ZR):
        pltpu.sync_copy(acc.at[pl.ds(base + q * ZR, ZR)],
                        rows0.at[pl.ds(0, ZR)])
        pltpu.sync_copy(rows0.at[pl.ds(0, ZR)],
                        out_hbm.at[c, pl.ds(base + q * ZR, ZR)])


BN = 400
BD = 5120


BM = 1024


def _mm_body(w_ref, xt_ref, o_ref):
    o_ref[...] = lax.dot_general(w_ref[...].astype(jnp.bfloat16),
                                 xt_ref[...].astype(jnp.bfloat16),
                                 (((0,), (0,)), ((), ())),
                                 preferred_element_type=jnp.float32)


def _matmul(xt, W1):
    return pl.pallas_call(
        _mm_body,
        grid=(-(-N // BM),),
        in_specs=[pl.BlockSpec((IN_F, HID), lambda i: (0, 0)),
                  pl.BlockSpec((IN_F, BM), lambda i: (0, i))],
        out_specs=pl.BlockSpec((HID, BM), lambda i: (0, i)),
        out_shape=jax.ShapeDtypeStruct((HID, N), jnp.float32),
    )(W1, xt)


def _dinv_body(d0_ref, d1_ref, o_ref):
    o_ref[...] = lax.rsqrt(d0_ref[...] + d1_ref[...] + 1.0)[None, :]


def _dinv(degp):
    nb = NPAD // BD
    return pl.pallas_call(
        _dinv_body,
        grid=(nb,),
        in_specs=[pl.BlockSpec((BD,), lambda i: (i,)),
                  pl.BlockSpec((BD,), lambda i, _nb=nb: (i + _nb,))],
        out_specs=pl.BlockSpec((1, BD), lambda i: (0, i)),
        out_shape=jax.ShapeDtypeStruct((1, NPAD), jnp.float32),
    )(degp, degp)


def _scale_body(h_ref, v_ref, o_ref):
    o_ref[...] = h_ref[...] * v_ref[...]


def _scale(ht, dinvr):
    return pl.pallas_call(
        _scale_body,
        grid=(-(-N // BM),),
        in_specs=[pl.BlockSpec((HID, BM), lambda i: (0, i)),
                  pl.BlockSpec((1, BM), lambda i: (0, i))],
        out_specs=pl.BlockSpec((HID, BM), lambda i: (0, i)),
        out_shape=jax.ShapeDtypeStruct((HID, N), jnp.float32),
    )(ht, dinvr)


def _mid_body(p_ref, h_ref, v_ref, b_ref, o_ref):
    v = v_ref[...]
    agg = v * (p_ref[0] + p_ref[1] + h_ref[...])
    g = jnp.maximum(agg + b_ref[...], 0.0)
    o_ref[...] = v * g


def _mid(p1t, hs1t, dinvr, b1c):
    return pl.pallas_call(
        _mid_body,
        grid=(-(-N // BM),),
        in_specs=[pl.BlockSpec((NC, HID, BM), lambda i: (0, 0, i)),
                  pl.BlockSpec((HID, BM), lambda i: (0, i)),
                  pl.BlockSpec((1, BM), lambda i: (0, i)),
                  pl.BlockSpec((HID, 1), lambda i: (0, 0))],
        out_specs=pl.BlockSpec((HID, BM), lambda i: (0, i)),
        out_shape=jax.ShapeDtypeStruct((HID, N), jnp.float32),
    )(p1t, hs1t, dinvr, b1c)


def _out_body(p_ref, h_ref, v_ref, w_ref, b_ref, o_ref):
    v = v_ref[...]
    t = v * (p_ref[0] + p_ref[1] + h_ref[...])
    logits = lax.dot_general(w_ref[...], t, (((0,), (0,)), ((), ())),
                             preferred_element_type=jnp.float32) + b_ref[...]
    ml = jnp.max(logits, axis=0, keepdims=True)
    lse = jnp.log(jnp.sum(jnp.exp(logits - ml), axis=0, keepdims=True))
    o_ref[...] = logits - ml - lse


def _outk(p2t, hs2t, dinvr, W2, b2c):
    return pl.pallas_call(
        _out_body,
        grid=(-(-N // BM),),
        in_specs=[pl.BlockSpec((NC, HID, BM), lambda i: (0, 0, i)),
                  pl.BlockSpec((HID, BM), lambda i: (0, i)),
                  pl.BlockSpec((1, BM), lambda i: (0, i)),
                  pl.BlockSpec((HID, CLS), lambda i: (0, 0)),
                  pl.BlockSpec((CLS, 1), lambda i: (0, 0))],
        out_specs=pl.BlockSpec((CLS, BM), lambda i: (0, i)),
        out_shape=jax.ShapeDtypeStruct((CLS, N), jnp.float32),
    )(p2t, hs2t, dinvr, W2, b2c)


def kernel(x, edge_index, W1, b1, W2, b2):
    src = edge_index[0]
    dst = edge_index[1]
    pad_n = EP - E
    pad_src = (jnp.arange(pad_n, dtype=jnp.int32) * 37) % N
    pad_dst = jnp.full((pad_n,), N, dtype=jnp.int32)
    src_p = jnp.concatenate([src, pad_src]).reshape(GROUPS, STREAM)
    dst_p = jnp.concatenate([dst, pad_dst]).reshape(GROUPS, STREAM)
    zeros1 = jnp.zeros((RPT,), jnp.float32)
    zeros2 = jnp.zeros((ZR, HID), jnp.float32)
    b1c = b1.reshape(HID, 1)
    b2c = b2.reshape(CLS, 1)

    h1t = _matmul(jnp.transpose(x), W1)
    degp = _deg_kernel(dst_p, zeros1)
    dinvr = _dinv(degp)
    hs1t = _scale(h1t, dinvr)
    hs1 = jnp.transpose(hs1t)
    p1 = _agg_kernel(src_p, dst_p, hs1, zeros2)
    hs2t = _mid(jnp.transpose(p1, (0, 2, 1)), hs1t, dinvr, b1c)
    hs2 = jnp.transpose(hs2t)
    p2 = _agg_kernel(src_p, dst_p, hs2, zeros2)
    outt = _outk(jnp.transpose(p2, (0, 2, 1)), hs2t, dinvr, W2, b2c)
    return jnp.transpose(outt)
